# Initial kernel scaffold; baseline (speedup 1.0000x reference)
import jax
import jax.numpy as jnp
from jax.experimental import pallas as pl


def kernel(input_ids, embed_weight):
    raise NotImplementedError("write your pallas kernel here")


# speedup vs baseline: 1.8332x; 1.8332x over previous
import functools

import jax
import jax.numpy as jnp
from jax import lax
from jax.experimental import pallas as pl
from jax.experimental.pallas import tpu as pltpu
from jax.experimental.pallas import tpu_sc as plsc

NC = 2
NS = 16
NW = NC * NS

K = 8
NBUF = 2


@functools.lru_cache(maxsize=None)
def _build(B, V, D):
    assert B % (NW * K) == 0
    bpw = B // NW
    chunks = bpw // K

    mesh = plsc.VectorSubcoreMesh(core_axis_name="c", subcore_axis_name="s")

    @functools.partial(
        pl.kernel,
        mesh=mesh,
        out_type=jax.ShapeDtypeStruct((B, D), jnp.float32),
        scratch_types=[
            pltpu.VMEM((bpw,), jnp.int32),
            pltpu.VMEM((NBUF, K, D), jnp.float32),
            pltpu.SemaphoreType.DMA,
            pltpu.SemaphoreType.DMA,
            pltpu.SemaphoreType.DMA,
            pltpu.SemaphoreType.DMA,
        ],
    )
    def emb(idx_hbm, tab_hbm, out_hbm, idx_v, bufs, gs0, gs1, ws0, ws1):
        gsems = [gs0, gs1]
        wsems = [ws0, ws1]
        wid = lax.axis_index("s") * NC + lax.axis_index("c")
        base = wid * bpw
        pltpu.sync_copy(idx_hbm.at[pl.ds(base, bpw)], idx_v)

        def start_gather(b, g):
            off = pl.multiple_of(g * K, K)
            pltpu.async_copy(
                tab_hbm.at[idx_v.at[pl.ds(off, K)]], bufs.at[b], gsems[b])

        def wait_gather(b):
            pltpu.make_async_copy(
                tab_hbm.at[idx_v.at[pl.ds(0, K)]], bufs.at[b], gsems[b]).wait()

        def start_write(b, g):
            off = pl.multiple_of(base + g * K, K)
            pltpu.async_copy(bufs.at[b], out_hbm.at[pl.ds(off, K)], wsems[b])

        def wait_write(b):
            pltpu.make_async_copy(
                bufs.at[b], out_hbm.at[pl.ds(0, K)], wsems[b]).wait()

        for b in range(NBUF):
            start_gather(b, b)

        @pl.loop(0, (chunks - NBUF) // NBUF)
        def _(i):
            for b in range(NBUF):
                g = i * NBUF + b
                wait_gather(b)
                start_write(b, g)
                wait_write(b)
                start_gather(b, g + NBUF)

        for b in range(NBUF):
            wait_gather(b)
            start_write(b, chunks - NBUF + b)
        for b in range(NBUF):
            wait_write(b)

    return emb


def kernel(input_ids, embed_weight):
    V, D = embed_weight.shape
    idx = input_ids.reshape(-1).astype(jnp.int32)
    B = idx.shape[0]
    out = _build(B, V, D)(idx, embed_weight)
    return out.reshape(input_ids.shape + (D,))
